# Initial kernel scaffold; baseline (speedup 1.0000x reference)
import jax
import jax.numpy as jnp
from jax.experimental import pallas as pl


def kernel(indices, table):
    raise NotImplementedError("write your pallas kernel here")


# speedup vs baseline: 1.1030x; 1.1030x over previous
import functools

import jax
import jax.numpy as jnp
from jax import lax
from jax.experimental import pallas as pl
from jax.experimental.pallas import tpu as pltpu
from jax.experimental.pallas import tpu_sc as plsc


@functools.lru_cache(maxsize=None)
def _build_gather(n_idx: int, dim: int):
    info = plsc.get_sparse_core_info()
    nw = info.num_cores * info.num_subcores
    assert n_idx % nw == 0
    bpw = n_idx // nw
    chunk = 1600
    while bpw % chunk != 0:
        chunk //= 2
    nchunk = bpw // chunk

    mesh = plsc.VectorSubcoreMesh(core_axis_name="c", subcore_axis_name="s")

    @functools.partial(
        pl.kernel,
        mesh=mesh,
        out_type=jax.ShapeDtypeStruct((n_idx, dim), jnp.float32),
        scratch_types=[
            pltpu.VMEM((chunk,), jnp.int32),
            pltpu.VMEM((chunk, dim), jnp.float32),
            pltpu.SemaphoreType.DMA,
        ],
        compiler_params=pltpu.CompilerParams(use_tc_tiling_on_sc=False),
    )
    def gather_kernel(idx_hbm, table_hbm, out_hbm, idx_v, rows_v, sem):
        wid = lax.axis_index("s") * info.num_cores + lax.axis_index("c")
        base = pl.multiple_of(wid * bpw, 8)

        def body(i, carry):
            off = pl.multiple_of(base + i * chunk, 8)
            pltpu.sync_copy(idx_hbm.at[pl.ds(off, chunk)], idx_v)
            pltpu.async_copy(table_hbm.at[idx_v], rows_v, sem).wait()
            pltpu.sync_copy(rows_v, out_hbm.at[pl.ds(off, chunk)])
            return carry

        lax.fori_loop(0, nchunk, body, 0)

    return gather_kernel


def kernel(indices, table):
    b, t = indices.shape
    dim = table.shape[1]
    flat = indices.reshape(-1).astype(jnp.int32)
    out = _build_gather(b * t, dim)(flat, table)
    return out.reshape(b, t, dim)


# speedup vs baseline: 1.1139x; 1.0099x over previous
import functools

import jax
import jax.numpy as jnp
from jax import lax
from jax.experimental import pallas as pl
from jax.experimental.pallas import tpu as pltpu
from jax.experimental.pallas import tpu_sc as plsc


@functools.lru_cache(maxsize=None)
def _build_gather(n_idx: int, dim: int):
    info = plsc.get_sparse_core_info()
    nw = info.num_cores * info.num_subcores
    assert n_idx % nw == 0
    bpw = n_idx // nw
    nbuf = 4
    chunk = 800
    while bpw % chunk != 0:
        chunk //= 2
    nchunk = bpw // chunk
    assert (nchunk - nbuf) % nbuf == 0

    mesh = plsc.VectorSubcoreMesh(core_axis_name="c", subcore_axis_name="s")

    @functools.partial(
        pl.kernel,
        mesh=mesh,
        out_type=jax.ShapeDtypeStruct((n_idx, dim), jnp.float32),
        scratch_types=[
            pltpu.VMEM((bpw,), jnp.int32),
            pltpu.VMEM((nbuf, chunk, dim), jnp.float32),
            pltpu.SemaphoreType.DMA((nbuf,)),
            pltpu.SemaphoreType.DMA((nbuf,)),
        ],
        compiler_params=pltpu.CompilerParams(use_tc_tiling_on_sc=False),
    )
    def gather_kernel(idx_hbm, table_hbm, out_hbm, idx_v, rows_v, gsem, ssem):
        wid = lax.axis_index("s") * info.num_cores + lax.axis_index("c")
        base = pl.multiple_of(wid * bpw, 8)
        pltpu.sync_copy(idx_hbm.at[pl.ds(base, bpw)], idx_v)

        def fire_gather(c, b):
            off = pl.multiple_of(c * chunk, 8)
            pltpu.async_copy(
                table_hbm.at[idx_v.at[pl.ds(off, chunk)]], rows_v.at[b], gsem.at[b]
            )

        def wait_gather(b):
            pltpu.make_async_copy(
                table_hbm.at[pl.ds(0, chunk)], rows_v.at[b], gsem.at[b]
            ).wait()

        def fire_store(c, b):
            off = pl.multiple_of(base + c * chunk, 8)
            pltpu.async_copy(rows_v.at[b], out_hbm.at[pl.ds(off, chunk)], ssem.at[b])

        def wait_store(b):
            pltpu.make_async_copy(
                rows_v.at[b], out_hbm.at[pl.ds(0, chunk)], ssem.at[b]
            ).wait()

        for b in range(nbuf):
            fire_gather(b, b)

        @pl.loop(0, nchunk - nbuf, step=nbuf)
        def _(i):
            for b in range(nbuf):
                c = i + b
                wait_gather(b)
                fire_store(c, b)
                wait_store(b)
                fire_gather(c + nbuf, b)

        for b in range(nbuf):
            wait_gather(b)
            fire_store(nchunk - nbuf + b, b)
        for b in range(nbuf):
            wait_store(b)

    return gather_kernel


def kernel(indices, table):
    b, t = indices.shape
    dim = table.shape[1]
    flat = indices.reshape(-1).astype(jnp.int32)
    out = _build_gather(b * t, dim)(flat, table)
    return out.reshape(b, t, dim)


# speedup vs baseline: 1.4659x; 1.3160x over previous
import functools

import jax
import jax.numpy as jnp
from jax import lax
from jax.experimental import pallas as pl
from jax.experimental.pallas import tpu as pltpu
from jax.experimental.pallas import tpu_sc as plsc


@functools.lru_cache(maxsize=None)
def _build_gather(n_tok: int, n_batch: int, dim: int):
    info = plsc.get_sparse_core_info()
    nlanes = info.num_lanes
    nw = info.num_cores * info.num_subcores
    assert n_batch % nw == 0
    chunk = n_batch // nw
    assert chunk % nlanes == 0

    mesh = plsc.VectorSubcoreMesh(core_axis_name="c", subcore_axis_name="s")

    @functools.partial(
        pl.kernel,
        mesh=mesh,
        out_type=jax.ShapeDtypeStruct((n_tok, dim, n_batch), jnp.float32),
        scratch_types=[
            pltpu.VMEM((2, chunk), jnp.int32),
            pltpu.VMEM((2, chunk, dim), jnp.float32),
            pltpu.VMEM((2, dim, chunk), jnp.float32),
            pltpu.SemaphoreType.DMA((2,)),
            pltpu.SemaphoreType.DMA((2,)),
        ],
        compiler_params=pltpu.CompilerParams(
            use_tc_tiling_on_sc=False, needs_layout_passes=False
        ),
    )
    def gather_kernel(idx_hbm, table_hbm, out_hbm, idx_v, rows_v, tbuf_v, gsem, ssem):
        wid = lax.axis_index("s") * info.num_cores + lax.axis_index("c")
        b0 = pl.multiple_of(wid * chunk, 128)

        def fire_gather(t, buf):
            pltpu.sync_copy(idx_hbm.at[t, pl.ds(b0, chunk)], idx_v.at[buf])
            pltpu.async_copy(
                table_hbm.at[idx_v.at[buf]], rows_v.at[buf], gsem.at[buf]
            )

        def wait_gather(buf):
            pltpu.make_async_copy(
                table_hbm.at[pl.ds(0, chunk)], rows_v.at[buf], gsem.at[buf]
            ).wait()

        def transpose(buf):
            lanes = lax.iota(jnp.int32, nlanes)

            @pl.loop(0, chunk // nlanes)
            def _(i):
                row_idx = lanes + i * nlanes
                for d in range(dim):
                    col_idx = jnp.full((nlanes,), d, jnp.int32)
                    vals = plsc.load_gather(rows_v.at[buf], [row_idx, col_idx])
                    tbuf_v[buf, d, pl.ds(i * nlanes, nlanes)] = vals

        def fire_store(t, buf):
            pltpu.async_copy(
                tbuf_v.at[buf], out_hbm.at[t, :, pl.ds(b0, chunk)], ssem.at[buf]
            )

        def wait_store(buf):
            pltpu.make_async_copy(
                tbuf_v.at[buf], out_hbm.at[0, :, pl.ds(0, chunk)], ssem.at[buf]
            ).wait()

        assert n_tok >= 4 and n_tok % 2 == 0

        for b in range(2):
            fire_gather(b, b)
        for b in range(2):
            wait_gather(b)
            transpose(b)
            fire_store(b, b)
            fire_gather(b + 2, b)

        @pl.loop(0, (n_tok - 4) // 2)
        def _(i):
            for b in range(2):
                t = 2 + 2 * i + b
                wait_gather(b)
                wait_store(b)
                transpose(b)
                fire_store(t, b)
                fire_gather(t + 2, b)

        for b in range(2):
            wait_gather(b)
            wait_store(b)
            transpose(b)
            fire_store(n_tok - 2 + b, b)
        for b in range(2):
            wait_store(b)

    return gather_kernel


def kernel(indices, table):
    b, t = indices.shape
    dim = table.shape[1]
    idx_t = jnp.swapaxes(indices, 0, 1).astype(jnp.int32)
    out = _build_gather(t, b, dim)(idx_t, table)
    return jnp.transpose(out, (2, 0, 1))


# speedup vs baseline: 1.7168x; 1.1712x over previous
import functools

import jax
import jax.numpy as jnp
from jax import lax
from jax.experimental import pallas as pl
from jax.experimental.pallas import tpu as pltpu
from jax.experimental.pallas import tpu_sc as plsc


@functools.lru_cache(maxsize=None)
def _build_gather(n_tok: int, n_batch: int, dim: int):
    info = plsc.get_sparse_core_info()
    nlanes = info.num_lanes
    nw = info.num_cores * info.num_subcores
    assert n_batch % nw == 0
    chunk = n_batch // nw
    assert chunk % nlanes == 0

    mesh = plsc.VectorSubcoreMesh(core_axis_name="c", subcore_axis_name="s")

    @functools.partial(
        pl.kernel,
        mesh=mesh,
        out_type=jax.ShapeDtypeStruct((n_tok, dim, n_batch), jnp.float32),
        scratch_types=[
            pltpu.VMEM((2, chunk), jnp.int32),
            pltpu.VMEM((2, chunk, dim), jnp.float32),
            pltpu.VMEM((2, dim, chunk), jnp.float32),
            pltpu.SemaphoreType.DMA((2,)),
            pltpu.SemaphoreType.DMA((2,)),
        ],
        compiler_params=pltpu.CompilerParams(
            use_tc_tiling_on_sc=False, needs_layout_passes=False
        ),
    )
    def gather_kernel(idx_hbm, table_hbm, out_hbm, idx_v, rows_v, tbuf_v, gsem, ssem):
        wid = lax.axis_index("s") * info.num_cores + lax.axis_index("c")
        b0 = pl.multiple_of(wid * chunk, 128)

        def fire_gather(t, buf):
            pltpu.sync_copy(idx_hbm.at[t, pl.ds(b0, chunk)], idx_v.at[buf])
            pltpu.async_copy(
                table_hbm.at[idx_v.at[buf]], rows_v.at[buf], gsem.at[buf]
            )

        def wait_gather(buf):
            pltpu.make_async_copy(
                table_hbm.at[pl.ds(0, chunk)], rows_v.at[buf], gsem.at[buf]
            ).wait()

        def transpose(buf):
            lanes = lax.iota(jnp.int32, nlanes)

            @plsc.parallel_loop(0, chunk // nlanes, unroll=4)
            def _(i):
                row_idx = lanes + i * nlanes
                for d in range(dim):
                    col_idx = jnp.full((nlanes,), d, jnp.int32)
                    vals = plsc.load_gather(rows_v.at[buf], [row_idx, col_idx])
                    tbuf_v[buf, d, pl.ds(i * nlanes, nlanes)] = vals

        def fire_store(t, buf):
            pltpu.async_copy(
                tbuf_v.at[buf], out_hbm.at[t, :, pl.ds(b0, chunk)], ssem.at[buf]
            )

        def wait_store(buf):
            pltpu.make_async_copy(
                tbuf_v.at[buf], out_hbm.at[0, :, pl.ds(0, chunk)], ssem.at[buf]
            ).wait()

        assert n_tok >= 4 and n_tok % 2 == 0

        for b in range(2):
            fire_gather(b, b)
        for b in range(2):
            wait_gather(b)
            transpose(b)
            fire_store(b, b)
            fire_gather(b + 2, b)

        @pl.loop(0, (n_tok - 4) // 2)
        def _(i):
            for b in range(2):
                t = 2 + 2 * i + b
                wait_gather(b)
                wait_store(b)
                transpose(b)
                fire_store(t, b)
                fire_gather(t + 2, b)

        for b in range(2):
            wait_gather(b)
            wait_store(b)
            transpose(b)
            fire_store(n_tok - 2 + b, b)
        for b in range(2):
            wait_store(b)

    return gather_kernel


def kernel(indices, table):
    b, t = indices.shape
    dim = table.shape[1]
    idx_t = jnp.swapaxes(indices, 0, 1).astype(jnp.int32)
    out = _build_gather(t, b, dim)(idx_t, table)
    return jnp.transpose(out, (2, 0, 1))


# speedup vs baseline: 1.7651x; 1.0281x over previous
import functools

import jax
import jax.numpy as jnp
from jax import lax
from jax.experimental import pallas as pl
from jax.experimental.pallas import tpu as pltpu
from jax.experimental.pallas import tpu_sc as plsc


@functools.lru_cache(maxsize=None)
def _build_gather(n_tok: int, n_batch: int, dim: int):
    info = plsc.get_sparse_core_info()
    nlanes = info.num_lanes
    nw = info.num_cores * info.num_subcores
    assert n_batch % nw == 0
    chunk = n_batch // nw
    assert chunk % nlanes == 0

    mesh = plsc.VectorSubcoreMesh(core_axis_name="c", subcore_axis_name="s")

    @functools.partial(
        pl.kernel,
        mesh=mesh,
        out_type=jax.ShapeDtypeStruct((n_tok, dim, n_batch), jnp.float32),
        scratch_types=[
            pltpu.VMEM((n_tok, chunk), jnp.int32),
            pltpu.VMEM((2, chunk, dim), jnp.float32),
            pltpu.VMEM((2, dim, chunk), jnp.float32),
            pltpu.SemaphoreType.DMA((2,)),
            pltpu.SemaphoreType.DMA((2,)),
        ],
        compiler_params=pltpu.CompilerParams(
            use_tc_tiling_on_sc=False, needs_layout_passes=False
        ),
    )
    def gather_kernel(idx_hbm, table_hbm, out_hbm, idx_v, rows_v, tbuf_v, gsem, ssem):
        wid = lax.axis_index("s") * info.num_cores + lax.axis_index("c")
        b0 = pl.multiple_of(wid * chunk, 128)

        pltpu.sync_copy(idx_hbm.at[:, pl.ds(b0, chunk)], idx_v)

        def fire_gather(t, buf):
            pltpu.async_copy(
                table_hbm.at[idx_v.at[t]], rows_v.at[buf], gsem.at[buf]
            )

        def wait_gather(buf):
            pltpu.make_async_copy(
                table_hbm.at[pl.ds(0, chunk)], rows_v.at[buf], gsem.at[buf]
            ).wait()

        def transpose(buf):
            lanes = lax.iota(jnp.int32, nlanes)

            @plsc.parallel_loop(0, chunk // nlanes, unroll=4)
            def _(i):
                row_idx = lanes + i * nlanes
                for d in range(dim):
                    col_idx = jnp.full((nlanes,), d, jnp.int32)
                    vals = plsc.load_gather(rows_v.at[buf], [row_idx, col_idx])
                    tbuf_v[buf, d, pl.ds(i * nlanes, nlanes)] = vals

        def fire_store(t, buf):
            pltpu.async_copy(
                tbuf_v.at[buf], out_hbm.at[t, :, pl.ds(b0, chunk)], ssem.at[buf]
            )

        def wait_store(buf):
            pltpu.make_async_copy(
                tbuf_v.at[buf], out_hbm.at[0, :, pl.ds(0, chunk)], ssem.at[buf]
            ).wait()

        assert n_tok >= 4 and n_tok % 2 == 0

        for b in range(2):
            fire_gather(b, b)
        for b in range(2):
            wait_gather(b)
            transpose(b)
            fire_store(b, b)
            fire_gather(b + 2, b)

        @pl.loop(0, (n_tok - 4) // 2)
        def _(i):
            for b in range(2):
                t = 2 + 2 * i + b
                wait_gather(b)
                wait_store(b)
                transpose(b)
                fire_store(t, b)
                fire_gather(t + 2, b)

        for b in range(2):
            wait_gather(b)
            wait_store(b)
            transpose(b)
            fire_store(n_tok - 2 + b, b)
        for b in range(2):
            wait_store(b)

    return gather_kernel


def kernel(indices, table):
    b, t = indices.shape
    dim = table.shape[1]
    idx_t = jnp.swapaxes(indices, 0, 1).astype(jnp.int32)
    out = _build_gather(t, b, dim)(idx_t, table)
    return jnp.transpose(out, (2, 0, 1))
